# baseline (speedup 1.0000x reference)
import functools

import jax
import jax.numpy as jnp
from jax import lax
from jax.experimental import pallas as pl
from jax.experimental.pallas import tpu as pltpu
from jax.experimental.pallas import tpu_sc as plsc

B = 1024
D_IN = 512
LATENT = 64
SOM_H = 32
SOM_W = 32
N_CODES = SOM_H * SOM_W
BM = 128
GRID = B // BM

_DOT = functools.partial(
    jnp.dot,
    preferred_element_type=jnp.float32,
    precision=lax.Precision.DEFAULT,
)
_DG_HI = functools.partial(
    lax.dot_general,
    preferred_element_type=jnp.float32,
    precision=lax.Precision.HIGHEST,
)
_DG_LO = functools.partial(
    lax.dot_general,
    preferred_element_type=jnp.float32,
    precision=lax.Precision.DEFAULT,
)


def _tc_body(x_ref, wet_ref, be_ref, wdq_ref, bdq_ref, wde_ref, bde_ref,
             e_ref, xe_ref, zet_ref, dist_ref, k_ref, nbrt_ref, cb_ref):
    i = pl.program_id(0)
    x = x_ref[:]
    wet = wet_ref[:]
    e2d = e_ref[:]
    z = _DG_LO(x, wet, (((1,), (1,)), ((), ()))) + be_ref[:][None, :]
    zet_ref[:] = _DG_LO(wet, x, (((1,), (1,)), ((), ()))) + be_ref[:][:, None]
    xe_ref[:] = _DOT(z, wde_ref[:]) + bde_ref[:][None, :]

    score = _DG_HI(z, e2d, (((1,), (1,)), ((), ())))
    ones = jnp.ones((1, LATENT), jnp.float32)
    ee = _DG_HI(ones, e2d * e2d, (((1,), (1,)), ((), ())))
    key = ee - 2.0 * score
    zz = jnp.sum(z * z, axis=1, keepdims=True)
    dist_ref[:] = key + zz

    m = jnp.min(key, axis=1, keepdims=True)
    iot = lax.broadcasted_iota(jnp.int32, (BM, N_CODES), 1)
    hit = jnp.where(key == m, iot, jnp.int32(N_CODES))
    kcol = jnp.min(hit, axis=1, keepdims=True)
    k_ref[:] = jnp.min(hit, axis=1)

    k1 = kcol >> 5
    k2 = kcol & 31
    none = jnp.full_like(kcol, -1)
    up_c = jnp.where(k1 < SOM_H - 1, kcol + SOM_W, none)
    dn_c = jnp.where(k1 > 0, kcol - SOM_W, none)
    lf_c = jnp.where(k2 > 0, kcol - 1, none)

    def sel_t(col):
        oh = (iot == col).astype(jnp.float32)
        return _DG_HI(e2d, oh, (((0,), (1,)), ((), ())))

    nbrt_ref[0] = sel_t(kcol)
    nbrt_ref[1] = sel_t(up_c)
    nbrt_ref[2] = sel_t(dn_c)
    nbrt_ref[3] = jnp.zeros((LATENT, BM), jnp.float32)
    nbrt_ref[4] = sel_t(lf_c)

    @pl.when(i == 0)
    def _():
        cb_ref[:] = _DOT(e_ref[:], wdq_ref[:]) + bdq_ref[:][None, :]


def _tc_forward(x, WeT, b_enc, W_dec_q, b_dec_q, W_dec_e, b_dec_e, e2d):
    full = lambda *s: pl.BlockSpec(s, lambda i: (0,) * len(s))
    return pl.pallas_call(
        _tc_body,
        grid=(GRID,),
        in_specs=[
            pl.BlockSpec((BM, D_IN), lambda i: (i, 0)),
            full(LATENT, D_IN),
            full(LATENT),
            full(LATENT, D_IN),
            full(D_IN),
            full(LATENT, D_IN),
            full(D_IN),
            full(N_CODES, LATENT),
        ],
        out_specs=[
            pl.BlockSpec((BM, D_IN), lambda i: (i, 0)),
            pl.BlockSpec((LATENT, BM), lambda i: (0, i)),
            pl.BlockSpec((BM, N_CODES), lambda i: (i, 0)),
            pl.BlockSpec((BM,), lambda i: (i,)),
            pl.BlockSpec((5, LATENT, BM), lambda i: (0, 0, i)),
            full(N_CODES, D_IN),
        ],
        out_shape=[
            jax.ShapeDtypeStruct((B, D_IN), jnp.float32),
            jax.ShapeDtypeStruct((LATENT, B), jnp.float32),
            jax.ShapeDtypeStruct((B, N_CODES), jnp.float32),
            jax.ShapeDtypeStruct((B,), jnp.int32),
            jax.ShapeDtypeStruct((5, LATENT, B), jnp.float32),
            jax.ShapeDtypeStruct((N_CODES, D_IN), jnp.float32),
        ],
    )(x, WeT, b_enc, W_dec_q, b_dec_q, W_dec_e, b_dec_e, e2d)


_NC = 2
_NS = 16
_NW = _NC * _NS
BPW = B // _NW


def _sc_body(k_hbm, cb_hbm, xq_hbm, kv, cbrows, sem):
    wid = lax.axis_index("s") * _NC + lax.axis_index("c")
    base = wid * BPW
    pltpu.sync_copy(k_hbm.at[pl.ds(base, BPW)], kv)
    pltpu.async_copy(cb_hbm.at[kv], cbrows, sem).wait()
    pltpu.sync_copy(cbrows, xq_hbm.at[pl.ds(base, BPW)])


@functools.lru_cache(maxsize=1)
def _make_sc_gather():
    return functools.partial(
        pl.kernel,
        out_type=jax.ShapeDtypeStruct((B, D_IN), jnp.float32),
        scratch_types=[
            pltpu.VMEM((BPW,), jnp.int32),
            pltpu.VMEM((BPW, D_IN), jnp.float32),
            pltpu.SemaphoreType.DMA,
        ],
        mesh=plsc.VectorSubcoreMesh(core_axis_name="c", subcore_axis_name="s"),
    )(_sc_body)


def kernel(x, W_enc, b_enc, W_dec_q, b_dec_q, W_dec_e, b_dec_e, embeddings):
    e2d = embeddings.reshape(N_CODES, LATENT)
    x_e, zet, z_dist_flat, k, nbrt, cb = _tc_forward(
        x, W_enc.T, b_enc, W_dec_q, b_dec_q, W_dec_e, b_dec_e, e2d)
    x_q = _make_sc_gather()(k, cb)
    z_e = zet.T
    z_q_neighbors = jnp.transpose(nbrt, (2, 0, 1))
    z_q = z_q_neighbors[:, 0, :]
    return (x_e, x_q, z_e, z_q, z_q_neighbors, k, z_dist_flat)


# speedup vs baseline: 1.0829x; 1.0829x over previous
import functools

import jax
import jax.numpy as jnp
from jax import lax
from jax.experimental import pallas as pl
from jax.experimental.pallas import tpu as pltpu
from jax.experimental.pallas import tpu_sc as plsc

B = 1024
D_IN = 512
LATENT = 64
SOM_H = 32
SOM_W = 32
N_CODES = SOM_H * SOM_W
BM = 128
GRID = B // BM

_DOT = functools.partial(
    jnp.dot,
    preferred_element_type=jnp.float32,
    precision=lax.Precision.DEFAULT,
)
_DG_HI = functools.partial(
    lax.dot_general,
    preferred_element_type=jnp.float32,
    precision=lax.Precision.HIGHEST,
)
_DG_LO = functools.partial(
    lax.dot_general,
    preferred_element_type=jnp.float32,
    precision=lax.Precision.DEFAULT,
)


def _tc_body(x_ref, wet_ref, be_ref, wdq_ref, bdq_ref, wde_ref, bde_ref,
             e_ref, xe_ref, zet_ref, dist_ref, k_ref, nbrt_ref, zqt_ref,
             cb_ref):
    i = pl.program_id(0)
    x = x_ref[:]
    wet = wet_ref[:]
    e2d = e_ref[:]
    z = _DG_LO(x, wet, (((1,), (1,)), ((), ()))) + be_ref[:][None, :]
    zet_ref[:] = _DG_LO(wet, x, (((1,), (1,)), ((), ()))) + be_ref[:][:, None]
    xe_ref[:] = _DOT(z, wde_ref[:]) + bde_ref[:][None, :]

    score = _DG_HI(z, e2d, (((1,), (1,)), ((), ())))
    ones = jnp.ones((1, LATENT), jnp.float32)
    ee = _DG_HI(ones, e2d * e2d, (((1,), (1,)), ((), ())))
    key = ee - 2.0 * score
    zz = jnp.sum(z * z, axis=1, keepdims=True)
    dist_ref[:] = key + zz

    m = jnp.min(key, axis=1, keepdims=True)
    iot = lax.broadcasted_iota(jnp.int32, (BM, N_CODES), 1)
    hit = jnp.where(key == m, iot, jnp.int32(N_CODES))
    kcol = jnp.min(hit, axis=1, keepdims=True)
    k_ref[:] = jnp.min(hit, axis=1)

    k1 = kcol >> 5
    k2 = kcol & 31
    none = jnp.full_like(kcol, -1)
    up_c = jnp.where(k1 < SOM_H - 1, kcol + SOM_W, none)
    dn_c = jnp.where(k1 > 0, kcol - SOM_W, none)
    lf_c = jnp.where(k2 > 0, kcol - 1, none)

    def sel_t(col):
        oh = (iot == col).astype(jnp.float32)
        rows = _DG_HI(oh, e2d, (((1,), (0,)), ((), ())))
        return rows.T

    zqt = sel_t(kcol)
    nbrt_ref[0] = zqt
    zqt_ref[:] = zqt
    nbrt_ref[1] = sel_t(up_c)
    nbrt_ref[2] = sel_t(dn_c)
    nbrt_ref[3] = jnp.zeros((LATENT, BM), jnp.float32)
    nbrt_ref[4] = sel_t(lf_c)

    @pl.when(i == 0)
    def _():
        cb_ref[:] = _DOT(e_ref[:], wdq_ref[:]) + bdq_ref[:][None, :]


def _tc_forward(x, WeT, b_enc, W_dec_q, b_dec_q, W_dec_e, b_dec_e, e2d):
    full = lambda *s: pl.BlockSpec(s, lambda i: (0,) * len(s))
    return pl.pallas_call(
        _tc_body,
        grid=(GRID,),
        in_specs=[
            pl.BlockSpec((BM, D_IN), lambda i: (i, 0)),
            full(LATENT, D_IN),
            full(LATENT),
            full(LATENT, D_IN),
            full(D_IN),
            full(LATENT, D_IN),
            full(D_IN),
            full(N_CODES, LATENT),
        ],
        out_specs=[
            pl.BlockSpec((BM, D_IN), lambda i: (i, 0)),
            pl.BlockSpec((LATENT, BM), lambda i: (0, i)),
            pl.BlockSpec((BM, N_CODES), lambda i: (i, 0)),
            pl.BlockSpec((BM,), lambda i: (i,)),
            pl.BlockSpec((5, LATENT, BM), lambda i: (0, 0, i)),
            pl.BlockSpec((LATENT, BM), lambda i: (0, i)),
            full(N_CODES, D_IN),
        ],
        out_shape=[
            jax.ShapeDtypeStruct((B, D_IN), jnp.float32),
            jax.ShapeDtypeStruct((LATENT, B), jnp.float32),
            jax.ShapeDtypeStruct((B, N_CODES), jnp.float32),
            jax.ShapeDtypeStruct((B,), jnp.int32),
            jax.ShapeDtypeStruct((5, LATENT, B), jnp.float32),
            jax.ShapeDtypeStruct((LATENT, B), jnp.float32),
            jax.ShapeDtypeStruct((N_CODES, D_IN), jnp.float32),
        ],
    )(x, WeT, b_enc, W_dec_q, b_dec_q, W_dec_e, b_dec_e, e2d)


_NC = 2
_NS = 16
_NW = _NC * _NS
BPW = B // _NW


def _sc_body(k_hbm, cb_hbm, xq_hbm, kv, cbrows, sem):
    wid = lax.axis_index("s") * _NC + lax.axis_index("c")
    base = wid * BPW
    pltpu.sync_copy(k_hbm.at[pl.ds(base, BPW)], kv)
    pltpu.async_copy(cb_hbm.at[kv], cbrows, sem).wait()
    pltpu.sync_copy(cbrows, xq_hbm.at[pl.ds(base, BPW)])


@functools.lru_cache(maxsize=1)
def _make_sc_gather():
    return functools.partial(
        pl.kernel,
        out_type=jax.ShapeDtypeStruct((B, D_IN), jnp.float32),
        scratch_types=[
            pltpu.VMEM((BPW,), jnp.int32),
            pltpu.VMEM((BPW, D_IN), jnp.float32),
            pltpu.SemaphoreType.DMA,
        ],
        mesh=plsc.VectorSubcoreMesh(core_axis_name="c", subcore_axis_name="s"),
    )(_sc_body)


def kernel(x, W_enc, b_enc, W_dec_q, b_dec_q, W_dec_e, b_dec_e, embeddings):
    e2d = embeddings.reshape(N_CODES, LATENT)
    x_e, zet, z_dist_flat, k, nbrt, zqt, cb = _tc_forward(
        x, W_enc.T, b_enc, W_dec_q, b_dec_q, W_dec_e, b_dec_e, e2d)
    x_q = _make_sc_gather()(k, cb)
    z_e = zet.T
    z_q = zqt.T
    z_q_neighbors = jnp.transpose(nbrt, (2, 0, 1))
    return (x_e, x_q, z_e, z_q, z_q_neighbors, k, z_dist_flat)


# speedup vs baseline: 1.1441x; 1.0566x over previous
import functools

import jax
import jax.numpy as jnp
from jax import lax
from jax.experimental import pallas as pl
from jax.experimental.pallas import tpu as pltpu
from jax.experimental.pallas import tpu_sc as plsc

B = 1024
D_IN = 512
LATENT = 64
SOM_H = 32
SOM_W = 32
N_CODES = SOM_H * SOM_W
BM = 128
GRID = B // BM

_DOT = functools.partial(
    jnp.dot,
    preferred_element_type=jnp.float32,
    precision=lax.Precision.DEFAULT,
)
_DG_HI = functools.partial(
    lax.dot_general,
    preferred_element_type=jnp.float32,
    precision=lax.Precision.HIGHEST,
)
_DG_LO = functools.partial(
    lax.dot_general,
    preferred_element_type=jnp.float32,
    precision=lax.Precision.DEFAULT,
)


def _tc_body(x_ref, wet_ref, be_ref, wdq_ref, bdq_ref, wde_ref, bde_ref,
             e_ref, xe_ref, zet_ref, dist_ref, k_ref, nbrt_ref, zqt_ref,
             cb_ref, ecat_ref):
    i = pl.program_id(0)

    @pl.when(i == 0)
    def _():
        ez = e_ref[:]
        zrows = jnp.zeros((SOM_W, LATENT), jnp.float32)
        e_up = jnp.concatenate([ez[SOM_W:], zrows], axis=0)
        e_dn = jnp.concatenate([zrows, ez[:N_CODES - SOM_W]], axis=0)
        e_lf = jnp.concatenate(
            [jnp.zeros((1, LATENT), jnp.float32), ez[:N_CODES - 1]], axis=0)
        keep = (lax.broadcasted_iota(jnp.int32, (N_CODES, 1), 0) & 31) != 0
        e_lf = jnp.where(keep, e_lf, 0.0)
        ecat_ref[:] = jnp.concatenate([ez, e_up, e_dn, e_lf], axis=1)
        cb_ref[:] = _DOT(e_ref[:], wdq_ref[:]) + bdq_ref[:][None, :]
    x = x_ref[:]
    wet = wet_ref[:]
    e2d = e_ref[:]
    z = _DG_LO(x, wet, (((1,), (1,)), ((), ()))) + be_ref[:][None, :]
    zet_ref[:] = _DG_LO(wet, x, (((1,), (1,)), ((), ()))) + be_ref[:][:, None]
    xe_ref[:] = _DOT(z, wde_ref[:]) + bde_ref[:][None, :]

    score = _DG_HI(z, e2d, (((1,), (1,)), ((), ())))
    ones = jnp.ones((1, LATENT), jnp.float32)
    ee = _DG_HI(ones, e2d * e2d, (((1,), (1,)), ((), ())))
    key = ee - 2.0 * score
    zz = jnp.sum(z * z, axis=1, keepdims=True)
    dist_ref[:] = key + zz

    m = jnp.min(key, axis=1, keepdims=True)
    iot = lax.broadcasted_iota(jnp.int32, (BM, N_CODES), 1)
    hit = jnp.where(key == m, iot, jnp.int32(N_CODES))
    kcol = jnp.min(hit, axis=1, keepdims=True)
    k_ref[:] = jnp.min(hit, axis=1)

    oh = (iot == kcol).astype(jnp.float32)
    rows4 = _DG_HI(oh, ecat_ref[:], (((1,), (0,)), ((), ())))
    zqt = rows4[:, 0:LATENT].T
    nbrt_ref[0] = zqt
    zqt_ref[:] = zqt
    nbrt_ref[1] = rows4[:, LATENT:2 * LATENT].T
    nbrt_ref[2] = rows4[:, 2 * LATENT:3 * LATENT].T
    nbrt_ref[3] = jnp.zeros((LATENT, BM), jnp.float32)
    nbrt_ref[4] = rows4[:, 3 * LATENT:4 * LATENT].T


def _tc_forward(x, WeT, b_enc, W_dec_q, b_dec_q, W_dec_e, b_dec_e, e2d):
    full = lambda *s: pl.BlockSpec(s, lambda i: (0,) * len(s))
    return pl.pallas_call(
        _tc_body,
        grid=(GRID,),
        in_specs=[
            pl.BlockSpec((BM, D_IN), lambda i: (i, 0)),
            full(LATENT, D_IN),
            full(LATENT),
            full(LATENT, D_IN),
            full(D_IN),
            full(LATENT, D_IN),
            full(D_IN),
            full(N_CODES, LATENT),
        ],
        out_specs=[
            pl.BlockSpec((BM, D_IN), lambda i: (i, 0)),
            pl.BlockSpec((LATENT, BM), lambda i: (0, i)),
            pl.BlockSpec((BM, N_CODES), lambda i: (i, 0)),
            pl.BlockSpec((BM,), lambda i: (i,)),
            pl.BlockSpec((5, LATENT, BM), lambda i: (0, 0, i)),
            pl.BlockSpec((LATENT, BM), lambda i: (0, i)),
            full(N_CODES, D_IN),
        ],
        out_shape=[
            jax.ShapeDtypeStruct((B, D_IN), jnp.float32),
            jax.ShapeDtypeStruct((LATENT, B), jnp.float32),
            jax.ShapeDtypeStruct((B, N_CODES), jnp.float32),
            jax.ShapeDtypeStruct((B,), jnp.int32),
            jax.ShapeDtypeStruct((5, LATENT, B), jnp.float32),
            jax.ShapeDtypeStruct((LATENT, B), jnp.float32),
            jax.ShapeDtypeStruct((N_CODES, D_IN), jnp.float32),
        ],
        scratch_shapes=[pltpu.VMEM((N_CODES, 4 * LATENT), jnp.float32)],
    )(x, WeT, b_enc, W_dec_q, b_dec_q, W_dec_e, b_dec_e, e2d)


_NC = 2
_NS = 16
_NW = _NC * _NS
BPW = B // _NW


def _sc_body(k_hbm, cb_hbm, xq_hbm, kv, cbrows, sem):
    wid = lax.axis_index("s") * _NC + lax.axis_index("c")
    base = wid * BPW
    pltpu.sync_copy(k_hbm.at[pl.ds(base, BPW)], kv)
    pltpu.async_copy(cb_hbm.at[kv], cbrows, sem).wait()
    pltpu.sync_copy(cbrows, xq_hbm.at[pl.ds(base, BPW)])


@functools.lru_cache(maxsize=1)
def _make_sc_gather():
    return functools.partial(
        pl.kernel,
        out_type=jax.ShapeDtypeStruct((B, D_IN), jnp.float32),
        scratch_types=[
            pltpu.VMEM((BPW,), jnp.int32),
            pltpu.VMEM((BPW, D_IN), jnp.float32),
            pltpu.SemaphoreType.DMA,
        ],
        mesh=plsc.VectorSubcoreMesh(core_axis_name="c", subcore_axis_name="s"),
    )(_sc_body)


def kernel(x, W_enc, b_enc, W_dec_q, b_dec_q, W_dec_e, b_dec_e, embeddings):
    e2d = embeddings.reshape(N_CODES, LATENT)
    x_e, zet, z_dist_flat, k, nbrt, zqt, cb = _tc_forward(
        x, W_enc.T, b_enc, W_dec_q, b_dec_q, W_dec_e, b_dec_e, e2d)
    x_q = _make_sc_gather()(k, cb)
    z_e = zet.T
    z_q = zqt.T
    z_q_neighbors = jnp.transpose(nbrt, (2, 0, 1))
    return (x_e, x_q, z_e, z_q, z_q_neighbors, k, z_dist_flat)


# speedup vs baseline: 1.3215x; 1.1550x over previous
import functools

import jax
import jax.numpy as jnp
from jax import lax
from jax.experimental import pallas as pl
from jax.experimental.pallas import tpu as pltpu
from jax.experimental.pallas import tpu_sc as plsc

B = 1024
D_IN = 512
LATENT = 64
SOM_H = 32
SOM_W = 32
N_CODES = SOM_H * SOM_W
BM = 256
GRID = B // BM

_DOT = functools.partial(
    jnp.dot,
    preferred_element_type=jnp.float32,
    precision=lax.Precision.DEFAULT,
)
_DG_HI = functools.partial(
    lax.dot_general,
    preferred_element_type=jnp.float32,
    precision=lax.Precision.HIGHEST,
)
_DG_LO = functools.partial(
    lax.dot_general,
    preferred_element_type=jnp.float32,
    precision=lax.Precision.DEFAULT,
)


def _tc_body(x_ref, wet_ref, be_ref, wdq_ref, bdq_ref, wde_ref, bde_ref,
             e_ref, xe_ref, zet_ref, dist_ref, k_ref, nbrt_ref, zqt_ref,
             cb_ref, ecat_ref, ee_ref):
    i = pl.program_id(0)

    @pl.when(i == 0)
    def _():
        ez = e_ref[:]
        zrows = jnp.zeros((SOM_W, LATENT), jnp.float32)
        e_up = jnp.concatenate([ez[SOM_W:], zrows], axis=0)
        e_dn = jnp.concatenate([zrows, ez[:N_CODES - SOM_W]], axis=0)
        e_lf = jnp.concatenate(
            [jnp.zeros((1, LATENT), jnp.float32), ez[:N_CODES - 1]], axis=0)
        keep = (lax.broadcasted_iota(jnp.int32, (N_CODES, 1), 0) & 31) != 0
        e_lf = jnp.where(keep, e_lf, 0.0)
        ecat_ref[:] = jnp.concatenate([ez, e_up, e_dn, e_lf], axis=1)
        cb_ref[:] = _DOT(e_ref[:], wdq_ref[:]) + bdq_ref[:][None, :]
        ones = jnp.ones((1, LATENT), jnp.float32)
        ee_ref[:] = _DG_HI(ones, ez * ez, (((1,), (1,)), ((), ())))
    x = x_ref[:]
    wet = wet_ref[:]
    e2d = e_ref[:]
    z = _DG_LO(x, wet, (((1,), (1,)), ((), ()))) + be_ref[:][None, :]
    zet_ref[:] = _DG_LO(wet, x, (((1,), (1,)), ((), ()))) + be_ref[:][:, None]
    xe_ref[:] = _DOT(z, wde_ref[:]) + bde_ref[:][None, :]

    score = _DG_HI(z, e2d, (((1,), (1,)), ((), ())))
    key = ee_ref[:] - 2.0 * score
    zz = jnp.sum(z * z, axis=1, keepdims=True)
    dist_ref[:] = key + zz

    m = jnp.min(key, axis=1, keepdims=True)
    iot = lax.broadcasted_iota(jnp.int32, (BM, N_CODES), 1)
    hit = jnp.where(key == m, iot, jnp.int32(N_CODES))
    kcol = jnp.min(hit, axis=1, keepdims=True)
    k_ref[:] = jnp.min(hit, axis=1)

    oh = (iot == kcol).astype(jnp.float32)
    rows4 = _DG_HI(oh, ecat_ref[:], (((1,), (0,)), ((), ())))
    zqt = rows4[:, 0:LATENT].T
    nbrt_ref[0] = zqt
    zqt_ref[:] = zqt
    nbrt_ref[1] = rows4[:, LATENT:2 * LATENT].T
    nbrt_ref[2] = rows4[:, 2 * LATENT:3 * LATENT].T
    nbrt_ref[3] = jnp.zeros((LATENT, BM), jnp.float32)
    nbrt_ref[4] = rows4[:, 3 * LATENT:4 * LATENT].T


def _tc_forward(x, WeT, b_enc, W_dec_q, b_dec_q, W_dec_e, b_dec_e, e2d):
    full = lambda *s: pl.BlockSpec(s, lambda i: (0,) * len(s))
    return pl.pallas_call(
        _tc_body,
        grid=(GRID,),
        in_specs=[
            pl.BlockSpec((BM, D_IN), lambda i: (i, 0)),
            full(LATENT, D_IN),
            full(LATENT),
            full(LATENT, D_IN),
            full(D_IN),
            full(LATENT, D_IN),
            full(D_IN),
            full(N_CODES, LATENT),
        ],
        out_specs=[
            pl.BlockSpec((BM, D_IN), lambda i: (i, 0)),
            pl.BlockSpec((LATENT, BM), lambda i: (0, i)),
            pl.BlockSpec((BM, N_CODES), lambda i: (i, 0)),
            pl.BlockSpec((BM,), lambda i: (i,)),
            pl.BlockSpec((5, LATENT, BM), lambda i: (0, 0, i)),
            pl.BlockSpec((LATENT, BM), lambda i: (0, i)),
            full(N_CODES, D_IN),
        ],
        out_shape=[
            jax.ShapeDtypeStruct((B, D_IN), jnp.float32),
            jax.ShapeDtypeStruct((LATENT, B), jnp.float32),
            jax.ShapeDtypeStruct((B, N_CODES), jnp.float32),
            jax.ShapeDtypeStruct((B,), jnp.int32),
            jax.ShapeDtypeStruct((5, LATENT, B), jnp.float32),
            jax.ShapeDtypeStruct((LATENT, B), jnp.float32),
            jax.ShapeDtypeStruct((N_CODES, D_IN), jnp.float32),
        ],
        scratch_shapes=[pltpu.VMEM((N_CODES, 4 * LATENT), jnp.float32),
                        pltpu.VMEM((1, N_CODES), jnp.float32)],
    )(x, WeT, b_enc, W_dec_q, b_dec_q, W_dec_e, b_dec_e, e2d)


_NC = 2
_NS = 16
_NW = _NC * _NS
BPW = B // _NW


def _sc_body(k_hbm, cb_hbm, xq_hbm, kv, cbrows, sem):
    wid = lax.axis_index("s") * _NC + lax.axis_index("c")
    base = wid * BPW
    pltpu.sync_copy(k_hbm.at[pl.ds(base, BPW)], kv)
    pltpu.async_copy(cb_hbm.at[kv], cbrows, sem).wait()
    pltpu.sync_copy(cbrows, xq_hbm.at[pl.ds(base, BPW)])


@functools.lru_cache(maxsize=1)
def _make_sc_gather():
    return functools.partial(
        pl.kernel,
        out_type=jax.ShapeDtypeStruct((B, D_IN), jnp.float32),
        scratch_types=[
            pltpu.VMEM((BPW,), jnp.int32),
            pltpu.VMEM((BPW, D_IN), jnp.float32),
            pltpu.SemaphoreType.DMA,
        ],
        mesh=plsc.VectorSubcoreMesh(core_axis_name="c", subcore_axis_name="s"),
    )(_sc_body)


def kernel(x, W_enc, b_enc, W_dec_q, b_dec_q, W_dec_e, b_dec_e, embeddings):
    e2d = embeddings.reshape(N_CODES, LATENT)
    x_e, zet, z_dist_flat, k, nbrt, zqt, cb = _tc_forward(
        x, W_enc.T, b_enc, W_dec_q, b_dec_q, W_dec_e, b_dec_e, e2d)
    x_q = _make_sc_gather()(k, cb)
    z_e = zet.T
    z_q = zqt.T
    z_q_neighbors = jnp.transpose(nbrt, (2, 0, 1))
    return (x_e, x_q, z_e, z_q, z_q_neighbors, k, z_dist_flat)


# speedup vs baseline: 1.3386x; 1.0130x over previous
import functools

import jax
import jax.numpy as jnp
from jax import lax
from jax.experimental import pallas as pl
from jax.experimental.pallas import tpu as pltpu
from jax.experimental.pallas import tpu_sc as plsc

B = 1024
D_IN = 512
LATENT = 64
SOM_H = 32
SOM_W = 32
N_CODES = SOM_H * SOM_W
BM = 512
GRID = B // BM

_DOT = functools.partial(
    jnp.dot,
    preferred_element_type=jnp.float32,
    precision=lax.Precision.DEFAULT,
)
_DG_HI = functools.partial(
    lax.dot_general,
    preferred_element_type=jnp.float32,
    precision=lax.Precision.HIGHEST,
)
_DG_LO = functools.partial(
    lax.dot_general,
    preferred_element_type=jnp.float32,
    precision=lax.Precision.DEFAULT,
)


def _tc_body(x_ref, wet_ref, be_ref, wdq_ref, bdq_ref, wde_ref, bde_ref,
             e_ref, xe_ref, zet_ref, dist_ref, k_ref, nbrt_ref, zqt_ref,
             cb_ref, ecat_ref, ee_ref):
    i = pl.program_id(0)

    @pl.when(i == 0)
    def _():
        ez = e_ref[:]
        zrows = jnp.zeros((SOM_W, LATENT), jnp.float32)
        e_up = jnp.concatenate([ez[SOM_W:], zrows], axis=0)
        e_dn = jnp.concatenate([zrows, ez[:N_CODES - SOM_W]], axis=0)
        e_lf = jnp.concatenate(
            [jnp.zeros((1, LATENT), jnp.float32), ez[:N_CODES - 1]], axis=0)
        keep = (lax.broadcasted_iota(jnp.int32, (N_CODES, 1), 0) & 31) != 0
        e_lf = jnp.where(keep, e_lf, 0.0)
        ecat_ref[:] = jnp.concatenate([ez, e_up, e_dn, e_lf], axis=1)
        cb_ref[:] = _DOT(e_ref[:], wdq_ref[:]) + bdq_ref[:][None, :]
        ones = jnp.ones((1, LATENT), jnp.float32)
        ee_ref[:] = _DG_HI(ones, ez * ez, (((1,), (1,)), ((), ())))
    x = x_ref[:]
    wet = wet_ref[:]
    e2d = e_ref[:]
    z = _DG_LO(x, wet, (((1,), (1,)), ((), ()))) + be_ref[:][None, :]
    zet_ref[:] = _DG_LO(wet, x, (((1,), (1,)), ((), ()))) + be_ref[:][:, None]
    xe_ref[:] = _DOT(z, wde_ref[:]) + bde_ref[:][None, :]

    score = _DG_HI(z, e2d, (((1,), (1,)), ((), ())))
    key = ee_ref[:] - 2.0 * score
    zz = jnp.sum(z * z, axis=1, keepdims=True)
    dist_ref[:] = key + zz

    m = jnp.min(key, axis=1, keepdims=True)
    iot = lax.broadcasted_iota(jnp.int32, (BM, N_CODES), 1)
    hit = jnp.where(key == m, iot, jnp.int32(N_CODES))
    kcol = jnp.min(hit, axis=1, keepdims=True)
    k_ref[:] = jnp.min(hit, axis=1)

    oh = (iot == kcol).astype(jnp.float32)
    rows4 = _DG_HI(oh, ecat_ref[:], (((1,), (0,)), ((), ())))
    zqt = rows4[:, 0:LATENT].T
    nbrt_ref[0] = zqt
    zqt_ref[:] = zqt
    nbrt_ref[1] = rows4[:, LATENT:2 * LATENT].T
    nbrt_ref[2] = rows4[:, 2 * LATENT:3 * LATENT].T
    nbrt_ref[3] = jnp.zeros((LATENT, BM), jnp.float32)
    nbrt_ref[4] = rows4[:, 3 * LATENT:4 * LATENT].T


def _tc_forward(x, WeT, b_enc, W_dec_q, b_dec_q, W_dec_e, b_dec_e, e2d):
    full = lambda *s: pl.BlockSpec(s, lambda i: (0,) * len(s))
    return pl.pallas_call(
        _tc_body,
        grid=(GRID,),
        in_specs=[
            pl.BlockSpec((BM, D_IN), lambda i: (i, 0)),
            full(LATENT, D_IN),
            full(LATENT),
            full(LATENT, D_IN),
            full(D_IN),
            full(LATENT, D_IN),
            full(D_IN),
            full(N_CODES, LATENT),
        ],
        out_specs=[
            pl.BlockSpec((BM, D_IN), lambda i: (i, 0)),
            pl.BlockSpec((LATENT, BM), lambda i: (0, i)),
            pl.BlockSpec((BM, N_CODES), lambda i: (i, 0)),
            pl.BlockSpec((BM,), lambda i: (i,)),
            pl.BlockSpec((5, LATENT, BM), lambda i: (0, 0, i)),
            pl.BlockSpec((LATENT, BM), lambda i: (0, i)),
            full(N_CODES, D_IN),
        ],
        out_shape=[
            jax.ShapeDtypeStruct((B, D_IN), jnp.float32),
            jax.ShapeDtypeStruct((LATENT, B), jnp.float32),
            jax.ShapeDtypeStruct((B, N_CODES), jnp.float32),
            jax.ShapeDtypeStruct((B,), jnp.int32),
            jax.ShapeDtypeStruct((5, LATENT, B), jnp.float32),
            jax.ShapeDtypeStruct((LATENT, B), jnp.float32),
            jax.ShapeDtypeStruct((N_CODES, D_IN), jnp.float32),
        ],
        scratch_shapes=[pltpu.VMEM((N_CODES, 4 * LATENT), jnp.float32),
                        pltpu.VMEM((1, N_CODES), jnp.float32)],
    )(x, WeT, b_enc, W_dec_q, b_dec_q, W_dec_e, b_dec_e, e2d)


_NC = 2
_NS = 16
_NW = _NC * _NS
BPW = B // _NW


def _sc_body(k_hbm, cb_hbm, xq_hbm, kv, cbrows, sem):
    wid = lax.axis_index("s") * _NC + lax.axis_index("c")
    base = wid * BPW
    pltpu.sync_copy(k_hbm.at[pl.ds(base, BPW)], kv)
    pltpu.async_copy(cb_hbm.at[kv], cbrows, sem).wait()
    pltpu.sync_copy(cbrows, xq_hbm.at[pl.ds(base, BPW)])


@functools.lru_cache(maxsize=1)
def _make_sc_gather():
    return functools.partial(
        pl.kernel,
        out_type=jax.ShapeDtypeStruct((B, D_IN), jnp.float32),
        scratch_types=[
            pltpu.VMEM((BPW,), jnp.int32),
            pltpu.VMEM((BPW, D_IN), jnp.float32),
            pltpu.SemaphoreType.DMA,
        ],
        mesh=plsc.VectorSubcoreMesh(core_axis_name="c", subcore_axis_name="s"),
    )(_sc_body)


def kernel(x, W_enc, b_enc, W_dec_q, b_dec_q, W_dec_e, b_dec_e, embeddings):
    e2d = embeddings.reshape(N_CODES, LATENT)
    x_e, zet, z_dist_flat, k, nbrt, zqt, cb = _tc_forward(
        x, W_enc.T, b_enc, W_dec_q, b_dec_q, W_dec_e, b_dec_e, e2d)
    x_q = _make_sc_gather()(k, cb)
    z_e = zet.T
    z_q = zqt.T
    z_q_neighbors = jnp.transpose(nbrt, (2, 0, 1))
    return (x_e, x_q, z_e, z_q, z_q_neighbors, k, z_dist_flat)


# speedup vs baseline: 1.3782x; 1.0296x over previous
import functools

import jax
import jax.numpy as jnp
from jax import lax
from jax.experimental import pallas as pl
from jax.experimental.pallas import tpu as pltpu
from jax.experimental.pallas import tpu_sc as plsc

B = 1024
D_IN = 512
LATENT = 64
SOM_H = 32
SOM_W = 32
N_CODES = SOM_H * SOM_W
BM = 1024
GRID = B // BM

_DOT = functools.partial(
    jnp.dot,
    preferred_element_type=jnp.float32,
    precision=lax.Precision.DEFAULT,
)
_DG_HI = functools.partial(
    lax.dot_general,
    preferred_element_type=jnp.float32,
    precision=lax.Precision.HIGHEST,
)
_DG_LO = functools.partial(
    lax.dot_general,
    preferred_element_type=jnp.float32,
    precision=lax.Precision.DEFAULT,
)


def _tc_body(x_ref, wet_ref, be_ref, wdq_ref, bdq_ref, wde_ref, bde_ref,
             e_ref, xe_ref, zet_ref, dist_ref, k_ref, nbrt_ref, zqt_ref,
             cb_ref, ecat_ref, ee_ref):
    i = pl.program_id(0)

    @pl.when(i == 0)
    def _():
        ez = e_ref[:]
        zrows = jnp.zeros((SOM_W, LATENT), jnp.float32)
        e_up = jnp.concatenate([ez[SOM_W:], zrows], axis=0)
        e_dn = jnp.concatenate([zrows, ez[:N_CODES - SOM_W]], axis=0)
        e_lf = jnp.concatenate(
            [jnp.zeros((1, LATENT), jnp.float32), ez[:N_CODES - 1]], axis=0)
        keep = (lax.broadcasted_iota(jnp.int32, (N_CODES, 1), 0) & 31) != 0
        e_lf = jnp.where(keep, e_lf, 0.0)
        ecat_ref[:] = jnp.concatenate([ez, e_up, e_dn, e_lf], axis=1)
        cb_ref[:] = _DOT(e_ref[:], wdq_ref[:]) + bdq_ref[:][None, :]
        ones = jnp.ones((1, LATENT), jnp.float32)
        ee_ref[:] = _DG_HI(ones, ez * ez, (((1,), (1,)), ((), ())))
    x = x_ref[:]
    wet = wet_ref[:]
    e2d = e_ref[:]
    z = _DG_LO(x, wet, (((1,), (1,)), ((), ()))) + be_ref[:][None, :]
    zet_ref[:] = _DG_LO(wet, x, (((1,), (1,)), ((), ()))) + be_ref[:][:, None]
    xe_ref[:] = _DOT(z, wde_ref[:]) + bde_ref[:][None, :]

    score = _DG_HI(z, e2d, (((1,), (1,)), ((), ())))
    key = ee_ref[:] - 2.0 * score
    zz = jnp.sum(z * z, axis=1, keepdims=True)
    dist_ref[:] = key + zz

    m = jnp.min(key, axis=1, keepdims=True)
    iot = lax.broadcasted_iota(jnp.int32, (BM, N_CODES), 1)
    hit = jnp.where(key == m, iot, jnp.int32(N_CODES))
    kcol = jnp.min(hit, axis=1, keepdims=True)
    k_ref[:] = jnp.min(hit, axis=1)

    oh = (iot == kcol).astype(jnp.float32)
    rows4 = _DG_HI(oh, ecat_ref[:], (((1,), (0,)), ((), ())))
    zqt = rows4[:, 0:LATENT].T
    nbrt_ref[0] = zqt
    zqt_ref[:] = zqt
    nbrt_ref[1] = rows4[:, LATENT:2 * LATENT].T
    nbrt_ref[2] = rows4[:, 2 * LATENT:3 * LATENT].T
    nbrt_ref[3] = jnp.zeros((LATENT, BM), jnp.float32)
    nbrt_ref[4] = rows4[:, 3 * LATENT:4 * LATENT].T


def _tc_forward(x, WeT, b_enc, W_dec_q, b_dec_q, W_dec_e, b_dec_e, e2d):
    full = lambda *s: pl.BlockSpec(s, lambda i: (0,) * len(s))
    return pl.pallas_call(
        _tc_body,
        grid=(GRID,),
        in_specs=[
            pl.BlockSpec((BM, D_IN), lambda i: (i, 0)),
            full(LATENT, D_IN),
            full(LATENT),
            full(LATENT, D_IN),
            full(D_IN),
            full(LATENT, D_IN),
            full(D_IN),
            full(N_CODES, LATENT),
        ],
        out_specs=[
            pl.BlockSpec((BM, D_IN), lambda i: (i, 0)),
            pl.BlockSpec((LATENT, BM), lambda i: (0, i)),
            pl.BlockSpec((BM, N_CODES), lambda i: (i, 0)),
            pl.BlockSpec((BM,), lambda i: (i,)),
            pl.BlockSpec((5, LATENT, BM), lambda i: (0, 0, i)),
            pl.BlockSpec((LATENT, BM), lambda i: (0, i)),
            full(N_CODES, D_IN),
        ],
        out_shape=[
            jax.ShapeDtypeStruct((B, D_IN), jnp.float32),
            jax.ShapeDtypeStruct((LATENT, B), jnp.float32),
            jax.ShapeDtypeStruct((B, N_CODES), jnp.float32),
            jax.ShapeDtypeStruct((B,), jnp.int32),
            jax.ShapeDtypeStruct((5, LATENT, B), jnp.float32),
            jax.ShapeDtypeStruct((LATENT, B), jnp.float32),
            jax.ShapeDtypeStruct((N_CODES, D_IN), jnp.float32),
        ],
        scratch_shapes=[pltpu.VMEM((N_CODES, 4 * LATENT), jnp.float32),
                        pltpu.VMEM((1, N_CODES), jnp.float32)],
    )(x, WeT, b_enc, W_dec_q, b_dec_q, W_dec_e, b_dec_e, e2d)


_NC = 2
_NS = 16
_NW = _NC * _NS
BPW = B // _NW


def _sc_body(k_hbm, cb_hbm, xq_hbm, kv, cbrows, sem):
    wid = lax.axis_index("s") * _NC + lax.axis_index("c")
    base = wid * BPW
    pltpu.sync_copy(k_hbm.at[pl.ds(base, BPW)], kv)
    pltpu.async_copy(cb_hbm.at[kv], cbrows, sem).wait()
    pltpu.sync_copy(cbrows, xq_hbm.at[pl.ds(base, BPW)])


@functools.lru_cache(maxsize=1)
def _make_sc_gather():
    return functools.partial(
        pl.kernel,
        out_type=jax.ShapeDtypeStruct((B, D_IN), jnp.float32),
        scratch_types=[
            pltpu.VMEM((BPW,), jnp.int32),
            pltpu.VMEM((BPW, D_IN), jnp.float32),
            pltpu.SemaphoreType.DMA,
        ],
        mesh=plsc.VectorSubcoreMesh(core_axis_name="c", subcore_axis_name="s"),
    )(_sc_body)


def kernel(x, W_enc, b_enc, W_dec_q, b_dec_q, W_dec_e, b_dec_e, embeddings):
    e2d = embeddings.reshape(N_CODES, LATENT)
    x_e, zet, z_dist_flat, k, nbrt, zqt, cb = _tc_forward(
        x, W_enc.T, b_enc, W_dec_q, b_dec_q, W_dec_e, b_dec_e, e2d)
    x_q = _make_sc_gather()(k, cb)
    z_e = zet.T
    z_q = zqt.T
    z_q_neighbors = jnp.transpose(nbrt, (2, 0, 1))
    return (x_e, x_q, z_e, z_q, z_q_neighbors, k, z_dist_flat)


# speedup vs baseline: 1.6816x; 1.2201x over previous
import functools

import jax
import jax.numpy as jnp
from jax import lax
from jax.experimental import pallas as pl
from jax.experimental.pallas import tpu as pltpu
from jax.experimental.pallas import tpu_sc as plsc

B = 1024
D_IN = 512
LATENT = 64
SOM_H = 32
SOM_W = 32
N_CODES = SOM_H * SOM_W

_DOT = functools.partial(
    jnp.dot,
    preferred_element_type=jnp.float32,
    precision=lax.Precision.DEFAULT,
)
_DG_HI = functools.partial(
    lax.dot_general,
    preferred_element_type=jnp.float32,
    precision=lax.Precision.HIGHEST,
)
_DG_LO = functools.partial(
    lax.dot_general,
    preferred_element_type=jnp.float32,
    precision=lax.Precision.DEFAULT,
)


def _tc_a_body(x_ref, wet_ref, be_ref, wdq_ref, bdq_ref, wde_ref, bde_ref,
               e_ref, xe_ref, zet_ref, dist_ref, k_ref, kcol_ref, cb_ref):
    x = x_ref[:]
    wet = wet_ref[:]
    e2d = e_ref[:]
    z = _DG_LO(x, wet, (((1,), (1,)), ((), ()))) + be_ref[:][None, :]
    zet_ref[:] = _DG_LO(wet, x, (((1,), (1,)), ((), ()))) + be_ref[:][:, None]
    xe_ref[:] = _DOT(z, wde_ref[:]) + bde_ref[:][None, :]
    cb_ref[:] = _DOT(e2d, wdq_ref[:]) + bdq_ref[:][None, :]

    score = _DG_HI(z, e2d, (((1,), (1,)), ((), ())))
    ones = jnp.ones((1, LATENT), jnp.float32)
    ee = _DG_HI(ones, e2d * e2d, (((1,), (1,)), ((), ())))
    key = ee - 2.0 * score
    zz = jnp.sum(z * z, axis=1, keepdims=True)
    dist_ref[:] = key + zz

    m = jnp.min(key, axis=1, keepdims=True)
    iot = lax.broadcasted_iota(jnp.int32, (B, N_CODES), 1)
    hit = jnp.where(key == m, iot, jnp.int32(N_CODES))
    kcol_ref[:] = jnp.min(hit, axis=1, keepdims=True)
    k_ref[:] = jnp.min(hit, axis=1)


def _tc_a(x, WeT, b_enc, W_dec_q, b_dec_q, W_dec_e, b_dec_e, e2d):
    return pl.pallas_call(
        _tc_a_body,
        out_shape=[
            jax.ShapeDtypeStruct((B, D_IN), jnp.float32),
            jax.ShapeDtypeStruct((LATENT, B), jnp.float32),
            jax.ShapeDtypeStruct((B, N_CODES), jnp.float32),
            jax.ShapeDtypeStruct((B,), jnp.int32),
            jax.ShapeDtypeStruct((B, 1), jnp.int32),
            jax.ShapeDtypeStruct((N_CODES, D_IN), jnp.float32),
        ],
    )(x, WeT, b_enc, W_dec_q, b_dec_q, W_dec_e, b_dec_e, e2d)


def _tc_b_body(kcol_ref, e_ref, nbrt_ref, zqt_ref):
    ez = e_ref[:]
    zrows = jnp.zeros((SOM_W, LATENT), jnp.float32)
    e_up = jnp.concatenate([ez[SOM_W:], zrows], axis=0)
    e_dn = jnp.concatenate([zrows, ez[:N_CODES - SOM_W]], axis=0)
    e_lf = jnp.concatenate(
        [jnp.zeros((1, LATENT), jnp.float32), ez[:N_CODES - 1]], axis=0)
    keep = (lax.broadcasted_iota(jnp.int32, (N_CODES, 1), 0) & 31) != 0
    e_lf = jnp.where(keep, e_lf, 0.0)
    ecat = jnp.concatenate([ez, e_up, e_dn, e_lf], axis=1)

    kcol = kcol_ref[:]
    iot = lax.broadcasted_iota(jnp.int32, (B, N_CODES), 1)
    oh = (iot == kcol).astype(jnp.float32)
    rows4 = _DG_HI(oh, ecat, (((1,), (0,)), ((), ())))
    zqt = rows4[:, 0:LATENT].T
    nbrt_ref[0] = zqt
    zqt_ref[:] = zqt
    nbrt_ref[1] = rows4[:, LATENT:2 * LATENT].T
    nbrt_ref[2] = rows4[:, 2 * LATENT:3 * LATENT].T
    nbrt_ref[3] = jnp.zeros((LATENT, B), jnp.float32)
    nbrt_ref[4] = rows4[:, 3 * LATENT:4 * LATENT].T


def _tc_b(kcol, e2d):
    return pl.pallas_call(
        _tc_b_body,
        out_shape=[
            jax.ShapeDtypeStruct((5, LATENT, B), jnp.float32),
            jax.ShapeDtypeStruct((LATENT, B), jnp.float32),
        ],
    )(kcol, e2d)


_NC = 2
_NS = 16
_NW = _NC * _NS
BPW = B // _NW


def _sc_body(k_hbm, cb_hbm, xq_hbm, kv, cbrows, sem):
    wid = lax.axis_index("s") * _NC + lax.axis_index("c")
    base = wid * BPW
    pltpu.sync_copy(k_hbm.at[pl.ds(base, BPW)], kv)
    pltpu.async_copy(cb_hbm.at[kv], cbrows, sem).wait()
    pltpu.sync_copy(cbrows, xq_hbm.at[pl.ds(base, BPW)])


@functools.lru_cache(maxsize=1)
def _make_sc_gather():
    return functools.partial(
        pl.kernel,
        out_type=jax.ShapeDtypeStruct((B, D_IN), jnp.float32),
        scratch_types=[
            pltpu.VMEM((BPW,), jnp.int32),
            pltpu.VMEM((BPW, D_IN), jnp.float32),
            pltpu.SemaphoreType.DMA,
        ],
        mesh=plsc.VectorSubcoreMesh(core_axis_name="c", subcore_axis_name="s"),
    )(_sc_body)


def kernel(x, W_enc, b_enc, W_dec_q, b_dec_q, W_dec_e, b_dec_e, embeddings):
    e2d = embeddings.reshape(N_CODES, LATENT)
    x_e, zet, z_dist_flat, k, kcol, cb = _tc_a(
        x, W_enc.T, b_enc, W_dec_q, b_dec_q, W_dec_e, b_dec_e, e2d)
    x_q = _make_sc_gather()(k, cb)
    nbrt, zqt = _tc_b(kcol, e2d)
    z_e = zet.T
    z_q = zqt.T
    z_q_neighbors = jnp.transpose(nbrt, (2, 0, 1))
    return (x_e, x_q, z_e, z_q, z_q_neighbors, k, z_dist_flat)
